# Initial kernel scaffold; baseline (speedup 1.0000x reference)
import jax
import jax.numpy as jnp
from jax.experimental import pallas as pl


def kernel(constraint_features, edge_indices, edge_features, variable_features, params):
    raise NotImplementedError("write your pallas kernel here")


# speedup vs baseline: 2.0942x; 2.0942x over previous
import functools

import jax
import jax.numpy as jnp
from jax import lax
from jax.experimental import pallas as pl
from jax.experimental.pallas import tpu as pltpu
from jax.experimental.pallas import tpu_sc as plsc

EMB = 64
N_NODE = 50000
N_EDGE = 800000
NC = 2
NS = 16
NW = NC * NS
CH = 128
NCHUNK = N_EDGE // CH
HALF = N_NODE // NC
SH_ROWS = 25600
ZR = 200
EPS = 1e-5


def _ln(x, g, b):
    mu = jnp.mean(x, axis=-1, keepdims=True)
    var = jnp.mean(jnp.square(x - mu), axis=-1, keepdims=True)
    return (x - mu) * jax.lax.rsqrt(var + EPS) * g + b


def _embed_body(x_ref, g_ref, b_ref, w1_ref, b1_ref, w2_ref, b2_ref, o_ref):
    x = _ln(x_ref[...], g_ref[...], b_ref[...])
    h = jax.nn.relu(jnp.dot(x, w1_ref[...].T) + b1_ref[...])
    o_ref[...] = jax.nn.relu(jnp.dot(h, w2_ref[...].T) + b2_ref[...])


def _embed(x, g, b, w1, b1, w2, b2, rows):
    n, f = x.shape
    grid = n // rows
    full = lambda *s: pl.BlockSpec(s, lambda i: (0,) * len(s))
    return pl.pallas_call(
        _embed_body,
        grid=(grid,),
        in_specs=[
            pl.BlockSpec((rows, f), lambda i: (i, 0)),
            full(f), full(f), full(EMB, f), full(EMB), full(EMB, EMB), full(EMB),
        ],
        out_specs=pl.BlockSpec((rows, EMB), lambda i: (i, 0)),
        out_shape=jax.ShapeDtypeStruct((n, EMB), jnp.float32),
    )(x, g, b, w1, b1, w2, b2)


def _pre_body(r_ref, l_ref, wl_ref, bl_ref, wr_ref, a_ref, b_ref):
    a_ref[...] = jnp.dot(r_ref[...], wl_ref[...].T) + bl_ref[...]
    b_ref[...] = jnp.dot(l_ref[...], wr_ref[...].T)


def _pre(right, left, wl, bl, wr, rows):
    n = right.shape[0]
    full = lambda *s: pl.BlockSpec(s, lambda i: (0,) * len(s))
    blk = pl.BlockSpec((rows, EMB), lambda i: (i, 0))
    return pl.pallas_call(
        _pre_body,
        grid=(n // rows,),
        in_specs=[blk, blk, full(EMB, EMB), full(EMB), full(EMB, EMB)],
        out_specs=[blk, blk],
        out_shape=[jax.ShapeDtypeStruct((n, EMB), jnp.float32)] * 2,
    )(right, left, wl, bl, wr)


def _edge_body(ga_ref, gb_ref, g_ref, b_ref, wf_ref, bf_ref, o_ref):
    joint = ga_ref[...] + gb_ref[...]
    m = jax.nn.relu(_ln(joint, g_ref[...], b_ref[...]))
    o_ref[...] = jnp.dot(m, wf_ref[...].T) + bf_ref[...]


def _edge_mlp(ga, gb, g, b, wf, bf, rows):
    e = ga.shape[0]
    full = lambda *s: pl.BlockSpec(s, lambda i: (0,) * len(s))
    blk = pl.BlockSpec((rows, EMB), lambda i: (i, 0))
    return pl.pallas_call(
        _edge_body,
        grid=(e // rows,),
        in_specs=[blk, blk, full(EMB), full(EMB), full(EMB, EMB), full(EMB)],
        out_specs=blk,
        out_shape=jax.ShapeDtypeStruct((e, EMB), jnp.float32),
    )(ga, gb, g, b, wf, bf)


def _post_body(agg_ref, r_ref, g_ref, b_ref, w1a_ref, w1b_ref, b1_ref,
               w2_ref, b2_ref, o_ref):
    h = _ln(agg_ref[...], g_ref[...], b_ref[...])
    z = jax.nn.relu(jnp.dot(h, w1a_ref[...].T) + jnp.dot(r_ref[...], w1b_ref[...].T)
                    + b1_ref[...])
    o_ref[...] = jnp.dot(z, w2_ref[...].T) + b2_ref[...]


def _post_head_body(agg_ref, r_ref, g_ref, b_ref, w1a_ref, w1b_ref, b1_ref,
                    w2_ref, b2_ref, hw_ref, hb_ref, o_ref):
    h = _ln(agg_ref[...], g_ref[...], b_ref[...])
    z = jax.nn.relu(jnp.dot(h, w1a_ref[...].T) + jnp.dot(r_ref[...], w1b_ref[...].T)
                    + b1_ref[...])
    v = jnp.dot(z, w2_ref[...].T) + b2_ref[...]
    o_ref[...] = jnp.sum(v * hw_ref[...], axis=-1, keepdims=True) + hb_ref[...]


def _post(agg, right, g, b, w1a, w1b, b1, w2, b2, rows, head=None):
    n = agg.shape[0]
    full = lambda *s: pl.BlockSpec(s, lambda i: (0,) * len(s))
    blk = pl.BlockSpec((rows, EMB), lambda i: (i, 0))
    ins = [blk, blk, full(EMB), full(EMB), full(EMB, EMB), full(EMB, EMB),
           full(EMB), full(EMB, EMB), full(EMB)]
    args = [agg, right, g, b, w1a, w1b, b1, w2, b2]
    if head is None:
        return pl.pallas_call(
            _post_body,
            grid=(n // rows,),
            in_specs=ins,
            out_specs=blk,
            out_shape=jax.ShapeDtypeStruct((n, EMB), jnp.float32),
        )(*args)
    hw, hb = head
    hb = hb.reshape(1, 1)
    return pl.pallas_call(
        _post_head_body,
        grid=(n // rows,),
        in_specs=ins + [full(1, EMB), full(1, 1)],
        out_specs=pl.BlockSpec((rows, 1), lambda i: (i, 0)),
        out_shape=jax.ShapeDtypeStruct((n, 1), jnp.float32),
    )(*(args + [hw, hb]))


_MESH = plsc.VectorSubcoreMesh(core_axis_name="c", subcore_axis_name="s",
                               num_cores=NC, num_subcores=NS)


def _gather_body(a_hbm, b_hbm, dst_hbm, src_hbm, ga_hbm, gb_hbm,
                 idx_d, idx_s, rows_a, rows_b, sem_a, sem_b):
    wid = lax.axis_index("s") * NC + lax.axis_index("c")

    def step(i, carry):
        k = wid + i * NW

        @pl.when(k < NCHUNK)
        def _():
            base = k * CH
            pltpu.sync_copy(dst_hbm.at[pl.ds(base, CH)], idx_d)
            pltpu.sync_copy(src_hbm.at[pl.ds(base, CH)], idx_s)
            cpa = pltpu.async_copy(a_hbm.at[idx_d], rows_a, sem_a)
            cpb = pltpu.async_copy(b_hbm.at[idx_s], rows_b, sem_b)
            cpa.wait()
            cpb.wait()
            pltpu.sync_copy(rows_a, ga_hbm.at[pl.ds(base, CH)])
            pltpu.sync_copy(rows_b, gb_hbm.at[pl.ds(base, CH)])

        return carry

    lax.fori_loop(0, (NCHUNK + NW - 1) // NW, step, 0)


@functools.partial(
    pl.kernel,
    out_type=[jax.ShapeDtypeStruct((N_EDGE, EMB), jnp.float32)] * 2,
    mesh=_MESH,
    scratch_types=[
        pltpu.VMEM((CH,), jnp.int32),
        pltpu.VMEM((CH,), jnp.int32),
        pltpu.VMEM((CH, EMB), jnp.float32),
        pltpu.VMEM((CH, EMB), jnp.float32),
        pltpu.SemaphoreType.DMA,
        pltpu.SemaphoreType.DMA,
    ],
    compiler_params=pltpu.CompilerParams(use_tc_tiling_on_sc=False),
)
def _sc_gather(*refs):
    _gather_body(*refs)


def _scatter_body(msg_hbm, dst_hbm, zeros_hbm, agg_hbm,
                  shared, idxb, loc, rows, outb):
    c = lax.axis_index("c")
    s = lax.axis_index("s")

    def zstep(j, carry):
        pltpu.sync_copy(zeros_hbm, shared.at[pl.ds((s * 8 + j) * ZR, ZR)])
        return carry

    lax.fori_loop(0, SH_ROWS // ZR // NS, zstep, 0)
    plsc.subcore_barrier()

    coff = c * HALF

    def step(i, carry):
        k = s + i * NS

        @pl.when(k < NCHUNK)
        def _():
            base = k * CH
            pltpu.sync_copy(dst_hbm.at[pl.ds(base, CH)], idxb)
            pltpu.sync_copy(msg_hbm.at[pl.ds(base, CH)], rows)
            for j in range(CH // 16):
                v = idxb[pl.ds(j * 16, 16)]
                li = v - coff
                ok = (li >= 0) & (li < HALF)
                loc[pl.ds(j * 16, 16)] = jnp.where(ok, li, HALF)
            pltpu.sync_copy(rows, shared.at[loc], add=True)

        return carry

    lax.fori_loop(0, (NCHUNK + NS - 1) // NS, step, 0)
    plsc.subcore_barrier()

    def ostep(i, carry):
        k = s + i * NS

        @pl.when(k < HALF // ZR)
        def _():
            pltpu.sync_copy(shared.at[pl.ds(k * ZR, ZR)], outb)
            pltpu.sync_copy(outb, agg_hbm.at[pl.ds(coff + k * ZR, ZR)])

        return carry

    lax.fori_loop(0, (HALF // ZR + NS - 1) // NS, ostep, 0)


@functools.partial(
    pl.kernel,
    out_type=jax.ShapeDtypeStruct((N_NODE, EMB), jnp.float32),
    mesh=_MESH,
    scratch_types=[
        pltpu.VMEM_SHARED((SH_ROWS, EMB), jnp.float32),
        pltpu.VMEM((CH,), jnp.int32),
        pltpu.VMEM((CH,), jnp.int32),
        pltpu.VMEM((CH, EMB), jnp.float32),
        pltpu.VMEM((ZR, EMB), jnp.float32),
    ],
    compiler_params=pltpu.CompilerParams(use_tc_tiling_on_sc=False),
)
def _sc_scatter(*refs):
    _scatter_body(*refs)


def _conv(p, pre, left, src, dst, right, zeros, head=None):
    a, b = _pre(right, left, p[pre + '_Wl'], p[pre + '_bl'], p[pre + '_Wr'],
                rows=2000)
    ga, gb = _sc_gather(a, b, dst, src)
    msg = _edge_mlp(ga, gb, p[pre + '_fin_ln_g'], p[pre + '_fin_ln_b'],
                    p[pre + '_Wfin'], p[pre + '_bfin'], rows=8000)
    agg = _sc_scatter(msg, dst, zeros)
    w1 = p[pre + '_Wo1']
    return _post(agg, right, p[pre + '_post_ln_g'], p[pre + '_post_ln_b'],
                 w1[:, :EMB], w1[:, EMB:], p[pre + '_bo1'],
                 p[pre + '_Wo2'], p[pre + '_bo2'], rows=2000, head=head)


def kernel(constraint_features, edge_indices, edge_features, variable_features,
           params):
    p = params
    del edge_features
    cons_idx = edge_indices[0].astype(jnp.int32)
    var_idx = edge_indices[1].astype(jnp.int32)
    zeros = jnp.zeros((ZR, EMB), jnp.float32)

    cons = _embed(constraint_features, p['cons_ln_g'], p['cons_ln_b'],
                  p['cons_W1'], p['cons_b1'], p['cons_W2'], p['cons_b2'],
                  rows=2000)
    var = _embed(variable_features, p['var_ln_g'], p['var_ln_b'],
                 p['var_W1'], p['var_b1'], p['var_W2'], p['var_b2'],
                 rows=2000)

    cons = _conv(p, 'v2c', var, var_idx, cons_idx, cons, zeros)
    out = _conv(p, 'c2v', cons, cons_idx, var_idx, var, zeros,
                head=(p['head_W'], p['head_b']))
    return out[:, 0]


# speedup vs baseline: 2.4370x; 1.1637x over previous
import functools

import jax
import jax.numpy as jnp
from jax import lax
from jax.experimental import pallas as pl
from jax.experimental.pallas import tpu as pltpu
from jax.experimental.pallas import tpu_sc as plsc

EMB = 64
N_NODE = 50000
N_EDGE = 800000
NC = 2
NS = 16
NW = NC * NS
CH = 128
NCHUNK = N_EDGE // CH
HALF = N_NODE // NC
SH_ROWS = 25200
ZR = 200
EPS = 1e-5


def _ln(x, g, b):
    mu = jnp.mean(x, axis=-1, keepdims=True)
    var = jnp.mean(jnp.square(x - mu), axis=-1, keepdims=True)
    return (x - mu) * jax.lax.rsqrt(var + EPS) * g + b


def _embed_body(x_ref, g_ref, b_ref, w1_ref, b1_ref, w2_ref, b2_ref, o_ref):
    x = _ln(x_ref[...], g_ref[...], b_ref[...])
    h = jax.nn.relu(jnp.dot(x, w1_ref[...].T) + b1_ref[...])
    o_ref[...] = jax.nn.relu(jnp.dot(h, w2_ref[...].T) + b2_ref[...])


def _embed(x, g, b, w1, b1, w2, b2, rows):
    n, f = x.shape
    grid = n // rows
    full = lambda *s: pl.BlockSpec(s, lambda i: (0,) * len(s))
    return pl.pallas_call(
        _embed_body,
        grid=(grid,),
        in_specs=[
            pl.BlockSpec((rows, f), lambda i: (i, 0)),
            full(f), full(f), full(EMB, f), full(EMB), full(EMB, EMB), full(EMB),
        ],
        out_specs=pl.BlockSpec((rows, EMB), lambda i: (i, 0)),
        out_shape=jax.ShapeDtypeStruct((n, EMB), jnp.float32),
    )(x, g, b, w1, b1, w2, b2)


def _pre_body(r_ref, l_ref, wl_ref, bl_ref, wr_ref, a_ref, b_ref):
    a_ref[...] = jnp.dot(r_ref[...], wl_ref[...].T) + bl_ref[...]
    b_ref[...] = jnp.dot(l_ref[...], wr_ref[...].T)


def _pre(right, left, wl, bl, wr, rows):
    n = right.shape[0]
    full = lambda *s: pl.BlockSpec(s, lambda i: (0,) * len(s))
    blk = pl.BlockSpec((rows, EMB), lambda i: (i, 0))
    return pl.pallas_call(
        _pre_body,
        grid=(n // rows,),
        in_specs=[blk, blk, full(EMB, EMB), full(EMB), full(EMB, EMB)],
        out_specs=[blk, blk],
        out_shape=[jax.ShapeDtypeStruct((n, EMB), jnp.float32)] * 2,
    )(right, left, wl, bl, wr)


def _edge_body(ga_ref, gb_ref, g_ref, b_ref, wf_ref, bf_ref, o_ref):
    joint = ga_ref[...] + gb_ref[...]
    m = jax.nn.relu(_ln(joint, g_ref[...], b_ref[...]))
    o_ref[...] = jnp.dot(m, wf_ref[...].T) + bf_ref[...]


def _edge_mlp(ga, gb, g, b, wf, bf, rows):
    e = ga.shape[0]
    full = lambda *s: pl.BlockSpec(s, lambda i: (0,) * len(s))
    blk = pl.BlockSpec((rows, EMB), lambda i: (i, 0))
    return pl.pallas_call(
        _edge_body,
        grid=(e // rows,),
        in_specs=[blk, blk, full(EMB), full(EMB), full(EMB, EMB), full(EMB)],
        out_specs=blk,
        out_shape=jax.ShapeDtypeStruct((e, EMB), jnp.float32),
    )(ga, gb, g, b, wf, bf)


def _post_body(agg_ref, r_ref, g_ref, b_ref, w1a_ref, w1b_ref, b1_ref,
               w2_ref, b2_ref, o_ref):
    h = _ln(agg_ref[...], g_ref[...], b_ref[...])
    z = jax.nn.relu(jnp.dot(h, w1a_ref[...].T) + jnp.dot(r_ref[...], w1b_ref[...].T)
                    + b1_ref[...])
    o_ref[...] = jnp.dot(z, w2_ref[...].T) + b2_ref[...]


def _post_head_body(agg_ref, r_ref, g_ref, b_ref, w1a_ref, w1b_ref, b1_ref,
                    w2_ref, b2_ref, hw_ref, hb_ref, o_ref):
    h = _ln(agg_ref[...], g_ref[...], b_ref[...])
    z = jax.nn.relu(jnp.dot(h, w1a_ref[...].T) + jnp.dot(r_ref[...], w1b_ref[...].T)
                    + b1_ref[...])
    v = jnp.dot(z, w2_ref[...].T) + b2_ref[...]
    o_ref[...] = jnp.sum(v * hw_ref[...], axis=-1, keepdims=True) + hb_ref[...]


def _post(agg, right, g, b, w1a, w1b, b1, w2, b2, rows, head=None):
    n = agg.shape[0]
    full = lambda *s: pl.BlockSpec(s, lambda i: (0,) * len(s))
    blk = pl.BlockSpec((rows, EMB), lambda i: (i, 0))
    ins = [blk, blk, full(EMB), full(EMB), full(EMB, EMB), full(EMB, EMB),
           full(EMB), full(EMB, EMB), full(EMB)]
    args = [agg, right, g, b, w1a, w1b, b1, w2, b2]
    if head is None:
        return pl.pallas_call(
            _post_body,
            grid=(n // rows,),
            in_specs=ins,
            out_specs=blk,
            out_shape=jax.ShapeDtypeStruct((n, EMB), jnp.float32),
        )(*args)
    hw, hb = head
    hb = hb.reshape(1, 1)
    return pl.pallas_call(
        _post_head_body,
        grid=(n // rows,),
        in_specs=ins + [full(1, EMB), full(1, 1)],
        out_specs=pl.BlockSpec((rows, 1), lambda i: (i, 0)),
        out_shape=jax.ShapeDtypeStruct((n, 1), jnp.float32),
    )(*(args + [hw, hb]))


_MESH = plsc.VectorSubcoreMesh(core_axis_name="c", subcore_axis_name="s",
                               num_cores=NC, num_subcores=NS)


def _gather_body(a_hbm, b_hbm, dst_hbm, src_hbm, ga_hbm, gb_hbm,
                 idx_d, idx_s, rows_a, rows_b, sem_g, sem_w):
    wid = lax.axis_index("s") * NC + lax.axis_index("c")
    iters = (NCHUNK + 2 * NW - 1) // (2 * NW)

    def step(i, carry):
        for b in range(2):
            k = wid + (2 * i + b) * NW

            @pl.when(k < NCHUNK)
            def _(b=b, k=k):
                base = k * CH

                @pl.when(i > 0)
                def _():
                    pltpu.make_async_copy(rows_a[b], ga_hbm.at[pl.ds(0, CH)],
                                          sem_w[b]).wait()
                    pltpu.make_async_copy(rows_b[b], gb_hbm.at[pl.ds(0, CH)],
                                          sem_w[b]).wait()

                pltpu.sync_copy(dst_hbm.at[pl.ds(base, CH)], idx_d[b])
                pltpu.sync_copy(src_hbm.at[pl.ds(base, CH)], idx_s[b])
                pltpu.async_copy(a_hbm.at[idx_d[b]], rows_a[b], sem_g[b])
                pltpu.async_copy(b_hbm.at[idx_s[b]], rows_b[b], sem_g[b])

        for b in range(2):
            k = wid + (2 * i + b) * NW

            @pl.when(k < NCHUNK)
            def _(b=b, k=k):
                base = k * CH
                pltpu.make_async_copy(a_hbm.at[idx_d[b]], rows_a[b],
                                      sem_g[b]).wait()
                pltpu.make_async_copy(b_hbm.at[idx_s[b]], rows_b[b],
                                      sem_g[b]).wait()
                pltpu.async_copy(rows_a[b], ga_hbm.at[pl.ds(base, CH)], sem_w[b])
                pltpu.async_copy(rows_b[b], gb_hbm.at[pl.ds(base, CH)], sem_w[b])

        return carry

    lax.fori_loop(0, iters, step, 0)

    for b in range(2):
        pltpu.make_async_copy(rows_a[b], ga_hbm.at[pl.ds(0, CH)],
                              sem_w[b]).wait()
        pltpu.make_async_copy(rows_b[b], gb_hbm.at[pl.ds(0, CH)],
                              sem_w[b]).wait()


@functools.partial(
    pl.kernel,
    out_type=[jax.ShapeDtypeStruct((N_EDGE, EMB), jnp.float32)] * 2,
    mesh=_MESH,
    scratch_types=[
        [pltpu.VMEM((CH,), jnp.int32)] * 2,
        [pltpu.VMEM((CH,), jnp.int32)] * 2,
        [pltpu.VMEM((CH, EMB), jnp.float32)] * 2,
        [pltpu.VMEM((CH, EMB), jnp.float32)] * 2,
        [pltpu.SemaphoreType.DMA] * 2,
        [pltpu.SemaphoreType.DMA] * 2,
    ],
    compiler_params=pltpu.CompilerParams(use_tc_tiling_on_sc=False),
)
def _sc_gather(*refs):
    _gather_body(*refs)


def _scatter_body(msg_hbm, dst_hbm, zeros_hbm, agg_hbm,
                  shared, idxb, loc, rows, outb, sem_r, sem_a):
    c = lax.axis_index("c")
    s = lax.axis_index("s")

    nz = SH_ROWS // ZR

    def zstep(j, carry):
        k = s + j * NS

        @pl.when(k < nz)
        def _():
            pltpu.sync_copy(zeros_hbm, shared.at[pl.ds(k * ZR, ZR)])

        return carry

    lax.fori_loop(0, (nz + NS - 1) // NS, zstep, 0)
    plsc.subcore_barrier()

    coff = c * HALF
    iters = (NCHUNK + 2 * NS - 1) // (2 * NS)

    def step(i, carry):
        for b in range(2):
            k = s + (2 * i + b) * NS

            @pl.when(k < NCHUNK)
            def _(b=b, k=k):
                base = k * CH

                @pl.when(i > 0)
                def _():
                    pltpu.make_async_copy(rows[b], shared.at[loc[b]],
                                          sem_a[b]).wait()

                pltpu.async_copy(dst_hbm.at[pl.ds(base, CH)], idxb[b], sem_r[b])
                pltpu.async_copy(msg_hbm.at[pl.ds(base, CH)], rows[b], sem_r[b])

        for b in range(2):
            k = s + (2 * i + b) * NS

            @pl.when(k < NCHUNK)
            def _(b=b, k=k):
                base = k * CH
                pltpu.make_async_copy(dst_hbm.at[pl.ds(base, CH)], idxb[b],
                                      sem_r[b]).wait()
                pltpu.make_async_copy(msg_hbm.at[pl.ds(base, CH)], rows[b],
                                      sem_r[b]).wait()
                for j in range(CH // 16):
                    v = idxb[b][pl.ds(j * 16, 16)]
                    li = v - coff
                    ok = (li >= 0) & (li < HALF)
                    loc[b][pl.ds(j * 16, 16)] = jnp.where(ok, li, HALF)
                pltpu.async_copy(rows[b], shared.at[loc[b]], sem_a[b], add=True)

        return carry

    lax.fori_loop(0, iters, step, 0)
    for b in range(2):
        pltpu.make_async_copy(rows[b], shared.at[loc[b]], sem_a[b]).wait()
    plsc.subcore_barrier()

    def ostep(i, carry):
        k = s + i * NS

        @pl.when(k < HALF // ZR)
        def _():
            pltpu.sync_copy(shared.at[pl.ds(k * ZR, ZR)], outb)
            pltpu.sync_copy(outb, agg_hbm.at[pl.ds(coff + k * ZR, ZR)])

        return carry

    lax.fori_loop(0, (HALF // ZR + NS - 1) // NS, ostep, 0)


@functools.partial(
    pl.kernel,
    out_type=jax.ShapeDtypeStruct((N_NODE, EMB), jnp.float32),
    mesh=_MESH,
    scratch_types=[
        pltpu.VMEM_SHARED((SH_ROWS, EMB), jnp.float32),
        [pltpu.VMEM((CH,), jnp.int32)] * 2,
        [pltpu.VMEM((CH,), jnp.int32)] * 2,
        [pltpu.VMEM((CH, EMB), jnp.float32)] * 2,
        pltpu.VMEM((ZR, EMB), jnp.float32),
        [pltpu.SemaphoreType.DMA] * 2,
        [pltpu.SemaphoreType.DMA] * 2,
    ],
    compiler_params=pltpu.CompilerParams(use_tc_tiling_on_sc=False),
)
def _sc_scatter(*refs):
    _scatter_body(*refs)


def _conv(p, pre, left, src, dst, right, zeros, head=None):
    a, b = _pre(right, left, p[pre + '_Wl'], p[pre + '_bl'], p[pre + '_Wr'],
                rows=2000)
    ga, gb = _sc_gather(a, b, dst, src)
    msg = _edge_mlp(ga, gb, p[pre + '_fin_ln_g'], p[pre + '_fin_ln_b'],
                    p[pre + '_Wfin'], p[pre + '_bfin'], rows=8000)
    agg = _sc_scatter(msg, dst, zeros)
    w1 = p[pre + '_Wo1']
    return _post(agg, right, p[pre + '_post_ln_g'], p[pre + '_post_ln_b'],
                 w1[:, :EMB], w1[:, EMB:], p[pre + '_bo1'],
                 p[pre + '_Wo2'], p[pre + '_bo2'], rows=2000, head=head)


def kernel(constraint_features, edge_indices, edge_features, variable_features,
           params):
    p = params
    del edge_features
    cons_idx = edge_indices[0].astype(jnp.int32)
    var_idx = edge_indices[1].astype(jnp.int32)
    zeros = jnp.zeros((ZR, EMB), jnp.float32)

    cons = _embed(constraint_features, p['cons_ln_g'], p['cons_ln_b'],
                  p['cons_W1'], p['cons_b1'], p['cons_W2'], p['cons_b2'],
                  rows=2000)
    var = _embed(variable_features, p['var_ln_g'], p['var_ln_b'],
                 p['var_W1'], p['var_b1'], p['var_W2'], p['var_b2'],
                 rows=2000)

    cons = _conv(p, 'v2c', var, var_idx, cons_idx, cons, zeros)
    out = _conv(p, 'c2v', cons, cons_idx, var_idx, var, zeros,
                head=(p['head_W'], p['head_b']))
    return out[:, 0]


# speedup vs baseline: 2.9099x; 1.1941x over previous
import functools

import jax
import jax.numpy as jnp
from jax import lax
from jax.experimental import pallas as pl
from jax.experimental.pallas import tpu as pltpu
from jax.experimental.pallas import tpu_sc as plsc

EMB = 64
N_NODE = 50000
N_EDGE = 800000
NC = 2
NS = 16
NW = NC * NS
CH = 128
NCHUNK = N_EDGE // CH
HALF = N_NODE // NC
SH_ROWS = 25200
ZR = 200
EPS = 1e-5


def _ln(x, g, b):
    mu = jnp.mean(x, axis=-1, keepdims=True)
    var = jnp.mean(jnp.square(x - mu), axis=-1, keepdims=True)
    return (x - mu) * jax.lax.rsqrt(var + EPS) * g + b


def _embed(x, g, b, w1, b1, w2, b2, rows, projs=()):
    n, f = x.shape
    full = lambda *s: pl.BlockSpec(s, lambda i: (0,) * len(s))
    blk = pl.BlockSpec((rows, EMB), lambda i: (i, 0))
    nproj = len(projs)
    biased = [pb is not None for (_, pb) in projs]

    def body(*refs):
        x_ref, g_ref, b_ref, w1_ref, b1_ref, w2_ref, b2_ref = refs[:7]
        wrefs = refs[7:7 + nproj + sum(biased)]
        orefs = refs[7 + nproj + sum(biased):]
        xv = _ln(x_ref[...], g_ref[...], b_ref[...])
        h = jax.nn.relu(jnp.dot(xv, w1_ref[...].T) + b1_ref[...])
        emb = jax.nn.relu(jnp.dot(h, w2_ref[...].T) + b2_ref[...])
        orefs[0][...] = emb
        wi = 0
        for j in range(nproj):
            y = jnp.dot(emb, wrefs[wi][...].T)
            wi += 1
            if biased[j]:
                y = y + wrefs[wi][...]
                wi += 1
            orefs[1 + j][...] = y

    in_specs = [pl.BlockSpec((rows, f), lambda i: (i, 0)),
                full(f), full(f), full(EMB, f), full(EMB),
                full(EMB, EMB), full(EMB)]
    args = [x, g, b, w1, b1, w2, b2]
    for (pw, pb) in projs:
        in_specs.append(full(EMB, EMB))
        args.append(pw)
        if pb is not None:
            in_specs.append(full(EMB))
            args.append(pb)
    return pl.pallas_call(
        body,
        grid=(n // rows,),
        in_specs=in_specs,
        out_specs=[blk] * (1 + nproj),
        out_shape=[jax.ShapeDtypeStruct((n, EMB), jnp.float32)] * (1 + nproj),
    )(*args)


def _edge_body(j_ref, g_ref, b_ref, wf_ref, bf_ref, o_ref):
    m = jax.nn.relu(_ln(j_ref[...], g_ref[...], b_ref[...]))
    o_ref[...] = jnp.dot(m, wf_ref[...].T) + bf_ref[...]


def _edge_mlp(joint, g, b, wf, bf, rows):
    e = joint.shape[0]
    full = lambda *s: pl.BlockSpec(s, lambda i: (0,) * len(s))
    blk = pl.BlockSpec((rows, EMB), lambda i: (i, 0))
    return pl.pallas_call(
        _edge_body,
        grid=(e // rows,),
        in_specs=[blk, full(EMB), full(EMB), full(EMB, EMB), full(EMB)],
        out_specs=blk,
        out_shape=jax.ShapeDtypeStruct((e, EMB), jnp.float32),
    )(joint, g, b, wf, bf)


def _post_body(agg_ref, r_ref, g_ref, b_ref, w1a_ref, w1b_ref, b1_ref,
               w2_ref, b2_ref, pw_ref, o_ref, p_ref):
    h = _ln(agg_ref[...], g_ref[...], b_ref[...])
    z = jax.nn.relu(jnp.dot(h, w1a_ref[...].T) + jnp.dot(r_ref[...], w1b_ref[...].T)
                    + b1_ref[...])
    out = jnp.dot(z, w2_ref[...].T) + b2_ref[...]
    o_ref[...] = out
    p_ref[...] = jnp.dot(out, pw_ref[...].T)


def _post_head_body(agg_ref, r_ref, g_ref, b_ref, w1a_ref, w1b_ref, b1_ref,
                    w2_ref, b2_ref, hw_ref, hb_ref, o_ref):
    h = _ln(agg_ref[...], g_ref[...], b_ref[...])
    z = jax.nn.relu(jnp.dot(h, w1a_ref[...].T) + jnp.dot(r_ref[...], w1b_ref[...].T)
                    + b1_ref[...])
    v = jnp.dot(z, w2_ref[...].T) + b2_ref[...]
    o_ref[...] = jnp.sum(v * hw_ref[...], axis=-1, keepdims=True) + hb_ref[...]


def _post(agg, right, g, b, w1a, w1b, b1, w2, b2, rows, head=None, proj_w=None):
    n = agg.shape[0]
    full = lambda *s: pl.BlockSpec(s, lambda i: (0,) * len(s))
    blk = pl.BlockSpec((rows, EMB), lambda i: (i, 0))
    ins = [blk, blk, full(EMB), full(EMB), full(EMB, EMB), full(EMB, EMB),
           full(EMB), full(EMB, EMB), full(EMB)]
    args = [agg, right, g, b, w1a, w1b, b1, w2, b2]
    if head is None:
        return pl.pallas_call(
            _post_body,
            grid=(n // rows,),
            in_specs=ins + [full(EMB, EMB)],
            out_specs=[blk, blk],
            out_shape=[jax.ShapeDtypeStruct((n, EMB), jnp.float32)] * 2,
        )(*(args + [proj_w]))
    hw, hb = head
    hb = hb.reshape(1, 1)
    return pl.pallas_call(
        _post_head_body,
        grid=(n // rows,),
        in_specs=ins + [full(1, EMB), full(1, 1)],
        out_specs=pl.BlockSpec((rows, 1), lambda i: (i, 0)),
        out_shape=jax.ShapeDtypeStruct((n, 1), jnp.float32),
    )(*(args + [hw, hb]))


_MESH = plsc.VectorSubcoreMesh(core_axis_name="c", subcore_axis_name="s",
                               num_cores=NC, num_subcores=NS)


def _gather_body(a_hbm, b_hbm, dst_hbm, src_hbm, jo_hbm,
                 idx_d, idx_s, rows_a, rows_b, sem_g, sem_w):
    wid = lax.axis_index("s") * NC + lax.axis_index("c")
    iters = (NCHUNK + 2 * NW - 1) // (2 * NW)

    def step(i, carry):
        for b in range(2):
            k = wid + (2 * i + b) * NW

            @pl.when(k < NCHUNK)
            def _(b=b, k=k):
                base = k * CH

                @pl.when(i > 0)
                def _():
                    pltpu.make_async_copy(rows_a[b], jo_hbm.at[pl.ds(0, CH)],
                                          sem_w[b]).wait()

                pltpu.sync_copy(dst_hbm.at[pl.ds(base, CH)], idx_d[b])
                pltpu.sync_copy(src_hbm.at[pl.ds(base, CH)], idx_s[b])
                pltpu.async_copy(a_hbm.at[idx_d[b]], rows_a[b], sem_g[b])
                pltpu.async_copy(b_hbm.at[idx_s[b]], rows_b[b], sem_g[b])

        for b in range(2):
            k = wid + (2 * i + b) * NW

            @pl.when(k < NCHUNK)
            def _(b=b, k=k):
                base = k * CH
                pltpu.make_async_copy(a_hbm.at[idx_d[b]], rows_a[b],
                                      sem_g[b]).wait()
                pltpu.make_async_copy(b_hbm.at[idx_s[b]], rows_b[b],
                                      sem_g[b]).wait()

                def add4(r, carry2, b=b):
                    for rr in range(4):
                        for jj in range(EMB // 16):
                            sl = pl.ds(jj * 16, 16)
                            rows_a[b][r * 4 + rr, sl] = (
                                rows_a[b][r * 4 + rr, sl]
                                + rows_b[b][r * 4 + rr, sl])
                    return carry2

                lax.fori_loop(0, CH // 4, add4, 0)
                pltpu.async_copy(rows_a[b], jo_hbm.at[pl.ds(base, CH)], sem_w[b])

        return carry

    lax.fori_loop(0, iters, step, 0)

    for b in range(2):
        pltpu.make_async_copy(rows_a[b], jo_hbm.at[pl.ds(0, CH)],
                              sem_w[b]).wait()


@functools.partial(
    pl.kernel,
    out_type=jax.ShapeDtypeStruct((N_EDGE, EMB), jnp.float32),
    mesh=_MESH,
    scratch_types=[
        [pltpu.VMEM((CH,), jnp.int32)] * 2,
        [pltpu.VMEM((CH,), jnp.int32)] * 2,
        [pltpu.VMEM((CH, EMB), jnp.float32)] * 2,
        [pltpu.VMEM((CH, EMB), jnp.float32)] * 2,
        [pltpu.SemaphoreType.DMA] * 2,
        [pltpu.SemaphoreType.DMA] * 2,
    ],
    compiler_params=pltpu.CompilerParams(use_tc_tiling_on_sc=False),
)
def _sc_gather(*refs):
    _gather_body(*refs)


def _scatter_body(msg_hbm, dst_hbm, zeros_hbm, agg_hbm,
                  shared, idxb, loc, rows, outb, sem_r, sem_a):
    c = lax.axis_index("c")
    s = lax.axis_index("s")

    nz = SH_ROWS // ZR

    def zstep(j, carry):
        k = s + j * NS

        @pl.when(k < nz)
        def _():
            pltpu.sync_copy(zeros_hbm, shared.at[pl.ds(k * ZR, ZR)])

        return carry

    lax.fori_loop(0, (nz + NS - 1) // NS, zstep, 0)
    plsc.subcore_barrier()

    coff = c * HALF
    iters = (NCHUNK + 2 * NS - 1) // (2 * NS)

    def step(i, carry):
        for b in range(2):
            k = s + (2 * i + b) * NS

            @pl.when(k < NCHUNK)
            def _(b=b, k=k):
                base = k * CH

                @pl.when(i > 0)
                def _():
                    pltpu.make_async_copy(rows[b], shared.at[loc[b]],
                                          sem_a[b]).wait()

                pltpu.async_copy(dst_hbm.at[pl.ds(base, CH)], idxb[b], sem_r[b])
                pltpu.async_copy(msg_hbm.at[pl.ds(base, CH)], rows[b], sem_r[b])

        for b in range(2):
            k = s + (2 * i + b) * NS

            @pl.when(k < NCHUNK)
            def _(b=b, k=k):
                base = k * CH
                pltpu.make_async_copy(dst_hbm.at[pl.ds(base, CH)], idxb[b],
                                      sem_r[b]).wait()
                pltpu.make_async_copy(msg_hbm.at[pl.ds(base, CH)], rows[b],
                                      sem_r[b]).wait()
                for j in range(CH // 16):
                    v = idxb[b][pl.ds(j * 16, 16)]
                    li = v - coff
                    ok = (li >= 0) & (li < HALF)
                    loc[b][pl.ds(j * 16, 16)] = jnp.where(ok, li, HALF)
                pltpu.async_copy(rows[b], shared.at[loc[b]], sem_a[b], add=True)

        return carry

    lax.fori_loop(0, iters, step, 0)
    for b in range(2):
        pltpu.make_async_copy(rows[b], shared.at[loc[b]], sem_a[b]).wait()
    plsc.subcore_barrier()

    def ostep(i, carry):
        k = s + i * NS

        @pl.when(k < HALF // ZR)
        def _():
            pltpu.sync_copy(shared.at[pl.ds(k * ZR, ZR)], outb)
            pltpu.sync_copy(outb, agg_hbm.at[pl.ds(coff + k * ZR, ZR)])

        return carry

    lax.fori_loop(0, (HALF // ZR + NS - 1) // NS, ostep, 0)


@functools.partial(
    pl.kernel,
    out_type=jax.ShapeDtypeStruct((N_NODE, EMB), jnp.float32),
    mesh=_MESH,
    scratch_types=[
        pltpu.VMEM_SHARED((SH_ROWS, EMB), jnp.float32),
        [pltpu.VMEM((CH,), jnp.int32)] * 2,
        [pltpu.VMEM((CH,), jnp.int32)] * 2,
        [pltpu.VMEM((CH, EMB), jnp.float32)] * 2,
        pltpu.VMEM((ZR, EMB), jnp.float32),
        [pltpu.SemaphoreType.DMA] * 2,
        [pltpu.SemaphoreType.DMA] * 2,
    ],
    compiler_params=pltpu.CompilerParams(use_tc_tiling_on_sc=False),
)
def _sc_scatter(*refs):
    _scatter_body(*refs)


def kernel(constraint_features, edge_indices, edge_features, variable_features,
           params):
    p = params
    del edge_features
    cons_idx = edge_indices[0].astype(jnp.int32)
    var_idx = edge_indices[1].astype(jnp.int32)
    zeros = jnp.zeros((ZR, EMB), jnp.float32)

    cons, a1 = _embed(constraint_features, p['cons_ln_g'], p['cons_ln_b'],
                      p['cons_W1'], p['cons_b1'], p['cons_W2'], p['cons_b2'],
                      rows=2000, projs=[(p['v2c_Wl'], p['v2c_bl'])])
    var, b1, a2 = _embed(variable_features, p['var_ln_g'], p['var_ln_b'],
                         p['var_W1'], p['var_b1'], p['var_W2'], p['var_b2'],
                         rows=2000,
                         projs=[(p['v2c_Wr'], None),
                                (p['c2v_Wl'], p['c2v_bl'])])

    joint1 = _sc_gather(a1, b1, cons_idx, var_idx)
    msg1 = _edge_mlp(joint1, p['v2c_fin_ln_g'], p['v2c_fin_ln_b'],
                     p['v2c_Wfin'], p['v2c_bfin'], rows=8000)
    agg1 = _sc_scatter(msg1, cons_idx, zeros)
    w1 = p['v2c_Wo1']
    cons2, b2 = _post(agg1, cons, p['v2c_post_ln_g'], p['v2c_post_ln_b'],
                      w1[:, :EMB], w1[:, EMB:], p['v2c_bo1'],
                      p['v2c_Wo2'], p['v2c_bo2'], rows=2000,
                      proj_w=p['c2v_Wr'])

    joint2 = _sc_gather(a2, b2, var_idx, cons_idx)
    msg2 = _edge_mlp(joint2, p['c2v_fin_ln_g'], p['c2v_fin_ln_b'],
                     p['c2v_Wfin'], p['c2v_bfin'], rows=8000)
    agg2 = _sc_scatter(msg2, var_idx, zeros)
    w2 = p['c2v_Wo1']
    out = _post(agg2, var, p['c2v_post_ln_g'], p['c2v_post_ln_b'],
                w2[:, :EMB], w2[:, EMB:], p['c2v_bo1'],
                p['c2v_Wo2'], p['c2v_bo2'], rows=2000,
                head=(p['head_W'], p['head_b']))
    return out[:, 0]


# speedup vs baseline: 3.2232x; 1.1077x over previous
import functools

import jax
import jax.numpy as jnp
from jax import lax
from jax.experimental import pallas as pl
from jax.experimental.pallas import tpu as pltpu
from jax.experimental.pallas import tpu_sc as plsc

EMB = 64
N_NODE = 50000
N_EDGE = 800000
NC = 2
NS = 16
NW = NC * NS
CH = 128
NB = 4
NBS = 2
NCHUNK = N_EDGE // CH
HALF = N_NODE // NC
SH_ROWS = 25200
ZR = 200
EPS = 1e-5


def _ln(x, g, b):
    mu = jnp.mean(x, axis=-1, keepdims=True)
    var = jnp.mean(jnp.square(x - mu), axis=-1, keepdims=True)
    return (x - mu) * jax.lax.rsqrt(var + EPS) * g + b


def _embed(x, g, b, w1, b1, w2, b2, rows, projs=()):
    n, f = x.shape
    full = lambda *s: pl.BlockSpec(s, lambda i: (0,) * len(s))
    blk = pl.BlockSpec((rows, EMB), lambda i: (i, 0))
    nproj = len(projs)
    biased = [pb is not None for (_, pb) in projs]

    def body(*refs):
        x_ref, g_ref, b_ref, w1_ref, b1_ref, w2_ref, b2_ref = refs[:7]
        wrefs = refs[7:7 + nproj + sum(biased)]
        orefs = refs[7 + nproj + sum(biased):]
        xv = _ln(x_ref[...], g_ref[...], b_ref[...])
        h = jax.nn.relu(jnp.dot(xv, w1_ref[...].T) + b1_ref[...])
        emb = jax.nn.relu(jnp.dot(h, w2_ref[...].T) + b2_ref[...])
        orefs[0][...] = emb
        wi = 0
        for j in range(nproj):
            y = jnp.dot(emb, wrefs[wi][...].T)
            wi += 1
            if biased[j]:
                y = y + wrefs[wi][...]
                wi += 1
            orefs[1 + j][...] = y

    in_specs = [pl.BlockSpec((rows, f), lambda i: (i, 0)),
                full(f), full(f), full(EMB, f), full(EMB),
                full(EMB, EMB), full(EMB)]
    args = [x, g, b, w1, b1, w2, b2]
    for (pw, pb) in projs:
        in_specs.append(full(EMB, EMB))
        args.append(pw)
        if pb is not None:
            in_specs.append(full(EMB))
            args.append(pb)
    return pl.pallas_call(
        body,
        grid=(n // rows,),
        in_specs=in_specs,
        out_specs=[blk] * (1 + nproj),
        out_shape=[jax.ShapeDtypeStruct((n, EMB), jnp.float32)] * (1 + nproj),
    )(*args)


def _edge_body(j_ref, g_ref, b_ref, wf_ref, bf_ref, o_ref):
    m = jax.nn.relu(_ln(j_ref[...], g_ref[...], b_ref[...]))
    o_ref[...] = jnp.dot(m, wf_ref[...].T) + bf_ref[...]


def _edge_mlp(joint, g, b, wf, bf, rows):
    e = joint.shape[0]
    full = lambda *s: pl.BlockSpec(s, lambda i: (0,) * len(s))
    blk = pl.BlockSpec((rows, EMB), lambda i: (i, 0))
    return pl.pallas_call(
        _edge_body,
        grid=(e // rows,),
        in_specs=[blk, full(EMB), full(EMB), full(EMB, EMB), full(EMB)],
        out_specs=blk,
        out_shape=jax.ShapeDtypeStruct((e, EMB), jnp.float32),
    )(joint, g, b, wf, bf)


def _post_body(agg_ref, r_ref, g_ref, b_ref, w1a_ref, w1b_ref, b1_ref,
               w2_ref, b2_ref, pw_ref, o_ref, p_ref):
    h = _ln(agg_ref[...], g_ref[...], b_ref[...])
    z = jax.nn.relu(jnp.dot(h, w1a_ref[...].T) + jnp.dot(r_ref[...], w1b_ref[...].T)
                    + b1_ref[...])
    out = jnp.dot(z, w2_ref[...].T) + b2_ref[...]
    o_ref[...] = out
    p_ref[...] = jnp.dot(out, pw_ref[...].T)


def _post_head_body(agg_ref, r_ref, g_ref, b_ref, w1a_ref, w1b_ref, b1_ref,
                    w2_ref, b2_ref, hw_ref, hb_ref, o_ref):
    h = _ln(agg_ref[...], g_ref[...], b_ref[...])
    z = jax.nn.relu(jnp.dot(h, w1a_ref[...].T) + jnp.dot(r_ref[...], w1b_ref[...].T)
                    + b1_ref[...])
    v = jnp.dot(z, w2_ref[...].T) + b2_ref[...]
    o_ref[...] = jnp.sum(v * hw_ref[...], axis=-1, keepdims=True) + hb_ref[...]


def _post(agg, right, g, b, w1a, w1b, b1, w2, b2, rows, head=None, proj_w=None):
    n = agg.shape[0]
    full = lambda *s: pl.BlockSpec(s, lambda i: (0,) * len(s))
    blk = pl.BlockSpec((rows, EMB), lambda i: (i, 0))
    ins = [blk, blk, full(EMB), full(EMB), full(EMB, EMB), full(EMB, EMB),
           full(EMB), full(EMB, EMB), full(EMB)]
    args = [agg, right, g, b, w1a, w1b, b1, w2, b2]
    if head is None:
        return pl.pallas_call(
            _post_body,
            grid=(n // rows,),
            in_specs=ins + [full(EMB, EMB)],
            out_specs=[blk, blk],
            out_shape=[jax.ShapeDtypeStruct((n, EMB), jnp.float32)] * 2,
        )(*(args + [proj_w]))
    hw, hb = head
    hb = hb.reshape(1, 1)
    return pl.pallas_call(
        _post_head_body,
        grid=(n // rows,),
        in_specs=ins + [full(1, EMB), full(1, 1)],
        out_specs=pl.BlockSpec((rows, 1), lambda i: (i, 0)),
        out_shape=jax.ShapeDtypeStruct((n, 1), jnp.float32),
    )(*(args + [hw, hb]))


_MESH = plsc.VectorSubcoreMesh(core_axis_name="c", subcore_axis_name="s",
                               num_cores=NC, num_subcores=NS)


def _gather_body(a_hbm, b_hbm, dst_hbm, src_hbm, jo_hbm,
                 idx_d, idx_s, rows_a, rows_b, sem_g, sem_w, sem_i):
    wid = lax.axis_index("s") * NC + lax.axis_index("c")
    iters = (NCHUNK + NB * NW - 1) // (NB * NW)

    def step(i, carry):
        for b in range(NB):
            k = wid + (NB * i + b) * NW

            @pl.when(k < NCHUNK)
            def _(b=b, k=k):
                base = k * CH

                @pl.when(i > 0)
                def _():
                    pltpu.make_async_copy(rows_a[b], jo_hbm.at[pl.ds(0, CH)],
                                          sem_w[b]).wait()

                pltpu.async_copy(dst_hbm.at[pl.ds(base, CH)], idx_d[b],
                                 sem_i[b])
                pltpu.async_copy(src_hbm.at[pl.ds(base, CH)], idx_s[b],
                                 sem_i[b])
                pltpu.make_async_copy(dst_hbm.at[pl.ds(base, CH)], idx_d[b],
                                      sem_i[b]).wait()
                pltpu.make_async_copy(src_hbm.at[pl.ds(base, CH)], idx_s[b],
                                      sem_i[b]).wait()
                pltpu.async_copy(a_hbm.at[idx_d[b]], rows_a[b], sem_g[b])
                pltpu.async_copy(b_hbm.at[idx_s[b]], rows_b[b], sem_g[b])

        for b in range(NB):
            k = wid + (NB * i + b) * NW

            @pl.when(k < NCHUNK)
            def _(b=b, k=k):
                base = k * CH
                pltpu.make_async_copy(a_hbm.at[idx_d[b]], rows_a[b],
                                      sem_g[b]).wait()
                pltpu.make_async_copy(b_hbm.at[idx_s[b]], rows_b[b],
                                      sem_g[b]).wait()

                def add4(r, carry2, b=b):
                    for rr in range(4):
                        for jj in range(EMB // 16):
                            sl = pl.ds(jj * 16, 16)
                            rows_a[b][r * 4 + rr, sl] = (
                                rows_a[b][r * 4 + rr, sl]
                                + rows_b[b][r * 4 + rr, sl])
                    return carry2

                lax.fori_loop(0, CH // 4, add4, 0)
                pltpu.async_copy(rows_a[b], jo_hbm.at[pl.ds(base, CH)], sem_w[b])

        return carry

    lax.fori_loop(0, iters, step, 0)

    for b in range(NB):
        pltpu.make_async_copy(rows_a[b], jo_hbm.at[pl.ds(0, CH)],
                              sem_w[b]).wait()


@functools.partial(
    pl.kernel,
    out_type=jax.ShapeDtypeStruct((N_EDGE, EMB), jnp.float32),
    mesh=_MESH,
    scratch_types=[
        [pltpu.VMEM((CH,), jnp.int32)] * NB,
        [pltpu.VMEM((CH,), jnp.int32)] * NB,
        [pltpu.VMEM((CH, EMB), jnp.float32)] * NB,
        [pltpu.VMEM((CH, EMB), jnp.float32)] * NB,
        [pltpu.SemaphoreType.DMA] * NB,
        [pltpu.SemaphoreType.DMA] * NB,
        [pltpu.SemaphoreType.DMA] * NB,
    ],
    compiler_params=pltpu.CompilerParams(use_tc_tiling_on_sc=False),
)
def _sc_gather(*refs):
    _gather_body(*refs)


def _scatter_body(msg_hbm, dst_hbm, zeros_hbm, agg_hbm,
                  shared, idxb, loc, rows, outb, sem_r, sem_a):
    c = lax.axis_index("c")
    s = lax.axis_index("s")

    nz = SH_ROWS // ZR

    def zstep(j, carry):
        k = s + j * NS

        @pl.when(k < nz)
        def _():
            pltpu.sync_copy(zeros_hbm, shared.at[pl.ds(k * ZR, ZR)])

        return carry

    lax.fori_loop(0, (nz + NS - 1) // NS, zstep, 0)
    plsc.subcore_barrier()

    coff = c * HALF
    iters = (NCHUNK + NBS * NS - 1) // (NBS * NS)

    def step(i, carry):
        for b in range(NBS):
            k = s + (NBS * i + b) * NS

            @pl.when(k < NCHUNK)
            def _(b=b, k=k):
                base = k * CH

                @pl.when(i > 0)
                def _():
                    pltpu.make_async_copy(rows[b], shared.at[loc[b]],
                                          sem_a[b]).wait()

                pltpu.async_copy(dst_hbm.at[pl.ds(base, CH)], idxb[b], sem_r[b])
                pltpu.async_copy(msg_hbm.at[pl.ds(base, CH)], rows[b], sem_r[b])

        for b in range(NBS):
            k = s + (NBS * i + b) * NS

            @pl.when(k < NCHUNK)
            def _(b=b, k=k):
                base = k * CH
                pltpu.make_async_copy(dst_hbm.at[pl.ds(base, CH)], idxb[b],
                                      sem_r[b]).wait()
                pltpu.make_async_copy(msg_hbm.at[pl.ds(base, CH)], rows[b],
                                      sem_r[b]).wait()
                for j in range(CH // 16):
                    v = idxb[b][pl.ds(j * 16, 16)]
                    li = v - coff
                    ok = (li >= 0) & (li < HALF)
                    loc[b][pl.ds(j * 16, 16)] = jnp.where(ok, li, HALF)
                pltpu.async_copy(rows[b], shared.at[loc[b]], sem_a[b], add=True)

        return carry

    lax.fori_loop(0, iters, step, 0)
    for b in range(NBS):
        pltpu.make_async_copy(rows[b], shared.at[loc[b]], sem_a[b]).wait()
    plsc.subcore_barrier()

    def ostep(i, carry):
        k = s + i * NS

        @pl.when(k < HALF // ZR)
        def _():
            pltpu.sync_copy(shared.at[pl.ds(k * ZR, ZR)], outb)
            pltpu.sync_copy(outb, agg_hbm.at[pl.ds(coff + k * ZR, ZR)])

        return carry

    lax.fori_loop(0, (HALF // ZR + NS - 1) // NS, ostep, 0)


@functools.partial(
    pl.kernel,
    out_type=jax.ShapeDtypeStruct((N_NODE, EMB), jnp.float32),
    mesh=_MESH,
    scratch_types=[
        pltpu.VMEM_SHARED((SH_ROWS, EMB), jnp.float32),
        [pltpu.VMEM((CH,), jnp.int32)] * NBS,
        [pltpu.VMEM((CH,), jnp.int32)] * NBS,
        [pltpu.VMEM((CH, EMB), jnp.float32)] * NBS,
        pltpu.VMEM((ZR, EMB), jnp.float32),
        [pltpu.SemaphoreType.DMA] * NBS,
        [pltpu.SemaphoreType.DMA] * NBS,
    ],
    compiler_params=pltpu.CompilerParams(use_tc_tiling_on_sc=False),
)
def _sc_scatter(*refs):
    _scatter_body(*refs)


def kernel(constraint_features, edge_indices, edge_features, variable_features,
           params):
    p = params
    del edge_features
    cons_idx = edge_indices[0].astype(jnp.int32)
    var_idx = edge_indices[1].astype(jnp.int32)
    zeros = jnp.zeros((ZR, EMB), jnp.float32)

    cons, a1 = _embed(constraint_features, p['cons_ln_g'], p['cons_ln_b'],
                      p['cons_W1'], p['cons_b1'], p['cons_W2'], p['cons_b2'],
                      rows=2000, projs=[(p['v2c_Wl'], p['v2c_bl'])])
    var, b1, a2 = _embed(variable_features, p['var_ln_g'], p['var_ln_b'],
                         p['var_W1'], p['var_b1'], p['var_W2'], p['var_b2'],
                         rows=2000,
                         projs=[(p['v2c_Wr'], None),
                                (p['c2v_Wl'], p['c2v_bl'])])

    joint1 = _sc_gather(a1, b1, cons_idx, var_idx)
    msg1 = _edge_mlp(joint1, p['v2c_fin_ln_g'], p['v2c_fin_ln_b'],
                     p['v2c_Wfin'], p['v2c_bfin'], rows=16000)
    agg1 = _sc_scatter(msg1, cons_idx, zeros)
    w1 = p['v2c_Wo1']
    cons2, b2 = _post(agg1, cons, p['v2c_post_ln_g'], p['v2c_post_ln_b'],
                      w1[:, :EMB], w1[:, EMB:], p['v2c_bo1'],
                      p['v2c_Wo2'], p['v2c_bo2'], rows=2000,
                      proj_w=p['c2v_Wr'])

    joint2 = _sc_gather(a2, b2, var_idx, cons_idx)
    msg2 = _edge_mlp(joint2, p['c2v_fin_ln_g'], p['c2v_fin_ln_b'],
                     p['c2v_Wfin'], p['c2v_bfin'], rows=16000)
    agg2 = _sc_scatter(msg2, var_idx, zeros)
    w2 = p['c2v_Wo1']
    out = _post(agg2, var, p['c2v_post_ln_g'], p['c2v_post_ln_b'],
                w2[:, :EMB], w2[:, EMB:], p['c2v_bo1'],
                p['c2v_Wo2'], p['c2v_bo2'], rows=2000,
                head=(p['head_W'], p['head_b']))
    return out[:, 0]
